# Initial kernel scaffold; baseline (speedup 1.0000x reference)
import jax
import jax.numpy as jnp
from jax.experimental import pallas as pl


def kernel(x, table, W1, b1, W2, b2):
    raise NotImplementedError("write your pallas kernel here")


# speedup vs baseline: 5.8942x; 5.8942x over previous
import functools

import jax
import jax.numpy as jnp
from jax import lax
from jax.experimental import pallas as pl
from jax.experimental.pallas import tpu as pltpu
from jax.experimental.pallas import tpu_sc as plsc

_LANES = 16
_GATHER = 128


@functools.partial(jax.jit, static_argnames=("n_samples", "seq_len"))
def _pool(x2d, table, n_samples, seq_len):
    info = plsc.get_sparse_core_info()
    nc, ns = info.num_cores, info.num_subcores
    nw = nc * ns
    samples_per_tile = n_samples // nw
    chunk_samples = 16
    chunks = samples_per_tile // chunk_samples
    chunk_idx = chunk_samples * seq_len
    n_gather = chunk_idx // _GATHER

    mesh = plsc.VectorSubcoreMesh(core_axis_name="c", subcore_axis_name="s")

    @functools.partial(
        pl.kernel,
        out_type=jax.ShapeDtypeStruct((n_samples, _LANES), jnp.float32),
        mesh=mesh,
        scratch_types=[
            pltpu.VMEM((n_gather, _GATHER), jnp.int32),
            pltpu.VMEM((chunk_idx, _LANES), jnp.float32),
            pltpu.VMEM((chunk_samples, _LANES), jnp.float32),
            pltpu.SemaphoreType.DMA,
        ],
        compiler_params=pltpu.CompilerParams(use_tc_tiling_on_sc=False),
    )
    def pool_kernel(x_hbm, table_hbm, out_hbm, idx_v, rows_v, iv_v, gsem):
        wid = lax.axis_index("s") * nc + lax.axis_index("c")
        samp0 = wid * samples_per_tile

        def chunk_body(ci, _):
            pltpu.sync_copy(x_hbm.at[wid * chunks + ci], idx_v)

            def fire(j, carry):
                pltpu.async_copy(table_hbm.at[idx_v.at[j]],
                                 rows_v.at[pl.ds(j * _GATHER, _GATHER)],
                                 gsem)
                return carry

            lax.fori_loop(0, n_gather, fire, 0)
            pltpu.make_async_copy(table_hbm.at[pl.ds(0, chunk_idx)],
                                  rows_v, gsem).wait()

            def sample_body(si, carry):
                base = si * seq_len

                def red(l, sc):
                    sv, cv = sc
                    v = rows_v[base + l]
                    sv = sv + v
                    cv = cv + jnp.where(v != 0.0, 1.0, 0.0)
                    return sv, cv

                z = jnp.zeros((_LANES,), jnp.float32)
                sv, cv = lax.fori_loop(0, seq_len, red, (z, z))
                iv_v[si] = sv / cv
                return carry

            lax.fori_loop(0, chunk_samples, sample_body, 0)
            pltpu.sync_copy(
                iv_v,
                out_hbm.at[pl.ds(samp0 + ci * chunk_samples, chunk_samples)])
            return _

        lax.fori_loop(0, chunks, chunk_body, 0)

    return pool_kernel(x2d, table)


def _mlp_body(iv_ref, w1_ref, b1_ref, w2_ref, b2_ref, out_ref):
    iv = iv_ref[...]
    h = lax.dot_general(iv, w1_ref[...], (((1,), (1,)), ((), ())),
                        preferred_element_type=jnp.float32)
    h = jnp.maximum(h + b1_ref[...], 0.0)
    o = lax.dot_general(h, w2_ref[...], (((1,), (1,)), ((), ())),
                        preferred_element_type=jnp.float32)
    out_ref[...] = o + b2_ref[...]


def _mlp(iv, W1, b1, W2, b2):
    n, d = iv.shape
    m = W1.shape[0]
    k = W2.shape[0]
    blk = 2048
    return pl.pallas_call(
        _mlp_body,
        grid=(n // blk,),
        in_specs=[
            pl.BlockSpec((blk, d), lambda i: (i, 0)),
            pl.BlockSpec((m, d), lambda i: (0, 0)),
            pl.BlockSpec((1, m), lambda i: (0, 0)),
            pl.BlockSpec((k, m), lambda i: (0, 0)),
            pl.BlockSpec((1, k), lambda i: (0, 0)),
        ],
        out_specs=pl.BlockSpec((blk, k), lambda i: (i, 0)),
        out_shape=jax.ShapeDtypeStruct((n, k), jnp.float32),
    )(iv, W1, b1.reshape(1, -1), W2, b2.reshape(1, -1))


def kernel(x, table, W1, b1, W2, b2):
    n_samples, seq_len = x.shape
    x3d = x.reshape(-1, (16 * seq_len) // _GATHER, _GATHER)
    iv = _pool(x3d, table, n_samples, seq_len)
    return _mlp(iv, W1, b1, W2, b2)


# speedup vs baseline: 9.4615x; 1.6052x over previous
import functools

import jax
import jax.numpy as jnp
from jax import lax
from jax.experimental import pallas as pl
from jax.experimental.pallas import tpu as pltpu
from jax.experimental.pallas import tpu_sc as plsc

_LANES = 16
_GATHER = 128


@functools.partial(jax.jit, static_argnames=("n_samples", "seq_len"))
def _pool(x2d, table, n_samples, seq_len):
    info = plsc.get_sparse_core_info()
    nc, ns = info.num_cores, info.num_subcores
    nw = nc * ns
    samples_per_tile = n_samples // nw
    chunk_samples = 16
    chunks = samples_per_tile // chunk_samples
    chunk_idx = chunk_samples * seq_len
    n_gather = chunk_idx // _GATHER

    mesh = plsc.VectorSubcoreMesh(core_axis_name="c", subcore_axis_name="s")

    unroll = 8
    red_iters = seq_len // unroll

    @functools.partial(
        pl.kernel,
        out_type=jax.ShapeDtypeStruct((n_samples, _LANES), jnp.float32),
        mesh=mesh,
        scratch_types=[
            pltpu.VMEM((2, n_gather, _GATHER), jnp.int32),
            pltpu.VMEM((2, chunk_idx, _LANES), jnp.float32),
            pltpu.VMEM((samples_per_tile, _LANES), jnp.float32),
            pltpu.SemaphoreType.DMA,
            pltpu.SemaphoreType.DMA,
        ],
        compiler_params=pltpu.CompilerParams(use_tc_tiling_on_sc=False),
    )
    def pool_kernel(x_hbm, table_hbm, out_hbm, idx_v, rows_v, iv_v,
                    sem0, sem1):
        wid = lax.axis_index("s") * nc + lax.axis_index("c")
        samp0 = wid * samples_per_tile
        chunk0 = wid * chunks

        def stage(ci, buf, sem):
            pltpu.sync_copy(x_hbm.at[chunk0 + ci], idx_v.at[buf])

            def fire(j, carry):
                pltpu.async_copy(table_hbm.at[idx_v.at[buf, j]],
                                 rows_v.at[buf, pl.ds(j * _GATHER, _GATHER)],
                                 sem)
                return carry

            lax.fori_loop(0, n_gather, fire, 0)

        def consume(ci, buf, sem):
            pltpu.make_async_copy(table_hbm.at[pl.ds(0, chunk_idx)],
                                  rows_v.at[buf], sem).wait()

            def sample_body(si, carry):
                base = si * seq_len
                out_slot = ci * chunk_samples + si

                def red(t, acc):
                    b = base + t * unroll
                    sv = list(acc[:4])
                    zv = list(acc[4:])
                    for k in range(unroll):
                        v = rows_v[buf, b + k]
                        sv[k % 4] = sv[k % 4] + v
                        zv[k % 4] = zv[k % 4] + jnp.where(v == 0.0, 1.0, 0.0)
                    return tuple(sv) + tuple(zv)

                z = jnp.zeros((_LANES,), jnp.float32)
                acc = lax.fori_loop(0, red_iters, red, (z,) * 8)
                sv = (acc[0] + acc[1]) + (acc[2] + acc[3])
                zv = (acc[4] + acc[5]) + (acc[6] + acc[7])
                iv_v[out_slot] = sv / (jnp.float32(seq_len) - zv)
                return carry

            lax.fori_loop(0, chunk_samples, sample_body, 0)

        stage(0, 0, sem0)

        def outer(g, carry):
            ci = 2 * g
            stage(ci + 1, 1, sem1)
            consume(ci, 0, sem0)

            @pl.when(ci + 2 < chunks)
            def _prefetch():
                stage(ci + 2, 0, sem0)

            consume(ci + 1, 1, sem1)
            return carry

        lax.fori_loop(0, chunks // 2, outer, 0)
        pltpu.sync_copy(iv_v,
                        out_hbm.at[pl.ds(samp0, samples_per_tile)])

    return pool_kernel(x2d, table)


def _mlp_body(iv_ref, w1_ref, b1_ref, w2_ref, b2_ref, out_ref):
    iv = iv_ref[...]
    h = lax.dot_general(iv, w1_ref[...], (((1,), (1,)), ((), ())),
                        preferred_element_type=jnp.float32)
    h = jnp.maximum(h + b1_ref[...], 0.0)
    o = lax.dot_general(h, w2_ref[...], (((1,), (1,)), ((), ())),
                        preferred_element_type=jnp.float32)
    out_ref[...] = o + b2_ref[...]


def _mlp(iv, W1, b1, W2, b2):
    n, d = iv.shape
    m = W1.shape[0]
    k = W2.shape[0]
    blk = 2048
    return pl.pallas_call(
        _mlp_body,
        grid=(n // blk,),
        in_specs=[
            pl.BlockSpec((blk, d), lambda i: (i, 0)),
            pl.BlockSpec((m, d), lambda i: (0, 0)),
            pl.BlockSpec((1, m), lambda i: (0, 0)),
            pl.BlockSpec((k, m), lambda i: (0, 0)),
            pl.BlockSpec((1, k), lambda i: (0, 0)),
        ],
        out_specs=pl.BlockSpec((blk, k), lambda i: (i, 0)),
        out_shape=jax.ShapeDtypeStruct((n, k), jnp.float32),
    )(iv, W1, b1.reshape(1, -1), W2, b2.reshape(1, -1))


def kernel(x, table, W1, b1, W2, b2):
    n_samples, seq_len = x.shape
    x3d = x.reshape(-1, (16 * seq_len) // _GATHER, _GATHER)
    iv = _pool(x3d, table, n_samples, seq_len)
    return _mlp(iv, W1, b1, W2, b2)
